# baseline (speedup 1.0000x reference)
import functools

import jax
import jax.numpy as jnp
from jax import lax
from jax.experimental import pallas as pl
from jax.experimental.pallas import tpu as pltpu
from jax.experimental.pallas import tpu_sc as plsc

_CH = 128
_NW = 32
_BN = 512


@functools.lru_cache(maxsize=None)
def _segsum_kernel(n_src, dc, e_pad, n_acc):
    cpw = e_pad // (_NW * _CH)
    rpt = n_acc // 16
    mesh = plsc.VectorSubcoreMesh(core_axis_name="c", subcore_axis_name="s")

    @functools.partial(
        pl.kernel, mesh=mesh,
        compiler_params=pltpu.CompilerParams(use_tc_tiling_on_sc=False),
        out_type=jax.ShapeDtypeStruct((2, n_acc, dc), jnp.float32),
        scratch_types=[
            pltpu.VMEM((_CH,), jnp.int32),
            pltpu.VMEM((_CH,), jnp.int32),
            pltpu.VMEM((_CH, dc), jnp.float32),
            pltpu.VMEM_SHARED((n_acc, dc), jnp.float32),
            pltpu.SemaphoreType.DMA,
        ],
    )
    def k(table_h, src_h, dst_h, zero_h, out_h, idx_v, dst_v, rows_v, acc, sem):
        cid = lax.axis_index("c")
        sid = lax.axis_index("s")
        wid = sid * 2 + cid
        pltpu.sync_copy(zero_h.at[pl.ds(sid * rpt, rpt)],
                        acc.at[pl.ds(sid * rpt, rpt)])
        plsc.subcore_barrier()

        def body(j, carry):
            base = wid * (cpw * _CH) + j * _CH
            pltpu.sync_copy(src_h.at[pl.ds(base, _CH)], idx_v)
            pltpu.sync_copy(dst_h.at[pl.ds(base, _CH)], dst_v)
            pltpu.async_copy(table_h.at[idx_v], rows_v, sem).wait()
            pltpu.sync_copy(rows_v, acc.at[dst_v], add=True)
            return carry

        lax.fori_loop(0, cpw, body, 0)
        plsc.subcore_barrier()
        pltpu.sync_copy(acc.at[pl.ds(sid * rpt, rpt)],
                        out_h.at[cid, pl.ds(sid * rpt, rpt)])

    return k


def _segsum_chunk(table, src_p, dst_p, n_dst, n_acc):
    n_src, dc = table.shape
    k = _segsum_kernel(n_src, dc, src_p.shape[0], n_acc)
    zeros = jnp.zeros((n_acc, dc), jnp.float32)
    out = k(table, src_p, dst_p, zeros)
    return (out[0] + out[1])[:n_dst]


def _segsum(table, edge, n_dst):
    src, dst = edge[0], edge[1]
    e = src.shape[0]
    per = _NW * _CH
    e_pad = -(-e // per) * per
    src_p = jnp.pad(src, (0, e_pad - e))
    dst_p = jnp.pad(dst, (0, e_pad - e), constant_values=n_dst)
    n_acc = -(-(n_dst + 1) // 128) * 128
    d = table.shape[1]
    dc = 16
    for c in (128, 64, 32, 16):
        if d % c == 0 and n_acc * c * 4 <= 6_500_000:
            dc = c
            break
    if dc == d:
        return _segsum_chunk(table, src_p, dst_p, n_dst, n_acc)
    parts = [_segsum_chunk(table[:, i * dc:(i + 1) * dc], src_p, dst_p,
                           n_dst, n_acc) for i in range(d // dc)]
    return jnp.concatenate(parts, axis=1)


def _counts(edge, n_dst):
    ones = jnp.ones((8, 16), jnp.float32)
    src0 = jnp.zeros_like(edge[0])
    return _segsum(ones, jnp.stack([src0, edge[1]]), n_dst)[:, 0]


@functools.lru_cache(maxsize=None)
def _dense_call(n_pad, k_dim, fuse):
    def body(a_ref, w_ref, b_ref, *rest):
        h = jnp.dot(a_ref[...], w_ref[...],
                    preferred_element_type=jnp.float32) + b_ref[...]
        h = jnp.maximum(h, 0.0)
        if fuse:
            w2_ref, b2_ref, o_ref = rest
            h = jnp.dot(h, w2_ref[...],
                        preferred_element_type=jnp.float32) + b2_ref[...]
        else:
            (o_ref,) = rest
        o_ref[...] = h

    in_specs = [
        pl.BlockSpec((_BN, k_dim), lambda i: (i, 0)),
        pl.BlockSpec((k_dim, 128), lambda i: (0, 0)),
        pl.BlockSpec((1, 128), lambda i: (0, 0)),
    ]
    if fuse:
        in_specs += [pl.BlockSpec((128, 128), lambda i: (0, 0)),
                     pl.BlockSpec((1, 128), lambda i: (0, 0))]
    return pl.pallas_call(
        body, grid=(n_pad // _BN,), in_specs=in_specs,
        out_specs=pl.BlockSpec((_BN, 128), lambda i: (i, 0)),
        out_shape=jax.ShapeDtypeStruct((n_pad, 128), jnp.float32))


def _dense(a, w, b, w2=None, b2=None):
    n, kd = a.shape
    n_pad = -(-n // _BN) * _BN
    a_p = jnp.pad(a, ((0, n_pad - n), (0, 0)))
    if w2 is None:
        out = _dense_call(n_pad, kd, False)(a_p, w, b.reshape(1, 128))
    else:
        out = _dense_call(n_pad, kd, True)(a_p, w, b.reshape(1, 128),
                                           w2, b2.reshape(1, 128))
    return out[:n]


def kernel(x_bus, x_generator, x_load, x_shunt, params,
           edge_index_ac_line, edge_index_transformer,
           edge_index_gen_to_bus, edge_index_bus_to_gen,
           edge_index_load_to_bus, edge_index_bus_to_load,
           edge_index_shunt_to_bus, edge_index_bus_to_shunt):
    nb = x_bus.shape[0]
    ng = x_generator.shape[0]
    nl = x_load.shape[0]
    ns = x_shunt.shape[0]

    dis_ac = lax.rsqrt(_counts(edge_index_ac_line, nb) + 1.0)
    dis_tr = lax.rsqrt(_counts(edge_index_transformer, nb) + 1.0)
    inv_g2b = 1.0 / jnp.maximum(_counts(edge_index_gen_to_bus, nb), 1.0)
    inv_l2b = 1.0 / jnp.maximum(_counts(edge_index_load_to_bus, nb), 1.0)
    inv_s2b = 1.0 / jnp.maximum(_counts(edge_index_shunt_to_bus, nb), 1.0)
    inv_b2g = 1.0 / jnp.maximum(_counts(edge_index_bus_to_gen, ng), 1.0)
    inv_b2l = 1.0 / jnp.maximum(_counts(edge_index_bus_to_load, nl), 1.0)
    inv_b2s = 1.0 / jnp.maximum(_counts(edge_index_bus_to_shunt, ns), 1.0)

    def layer(xb, xg, xl, xs_, lp):
        agg_ac = _segsum(xb * dis_ac[:, None], edge_index_ac_line, nb)
        f_ac = dis_ac[:, None] * (agg_ac + dis_ac[:, None] * xb)
        agg_tr = _segsum(xb * dis_tr[:, None], edge_index_transformer, nb)
        f_tr = dis_tr[:, None] * (agg_tr + dis_tr[:, None] * xb)
        m_g2b = _segsum(xg, edge_index_gen_to_bus, nb) * inv_g2b[:, None]
        m_l2b = _segsum(xl, edge_index_load_to_bus, nb) * inv_l2b[:, None]
        m_s2b = _segsum(xs_, edge_index_shunt_to_bus, nb) * inv_s2b[:, None]
        cat_b = jnp.concatenate([f_ac, f_tr, m_g2b, m_l2b, m_s2b, xb], axis=1)
        w_b = jnp.concatenate([
            lp["ac_line"]["W"], lp["transformer"]["W"],
            lp["gen2bus"]["Wl"], lp["load2bus"]["Wl"], lp["shunt2bus"]["Wl"],
            lp["gen2bus"]["Wr"] + lp["load2bus"]["Wr"] + lp["shunt2bus"]["Wr"],
        ], axis=0)
        b_b = (lp["ac_line"]["b"] + lp["transformer"]["b"]
               + lp["gen2bus"]["bl"] + lp["load2bus"]["bl"]
               + lp["shunt2bus"]["bl"])

        def sage_dst(x_dst, rel, edge, n_dst, inv):
            m = _segsum(xb, edge, n_dst) * inv[:, None]
            cat = jnp.concatenate([m, x_dst], axis=1)
            w = jnp.concatenate([lp[rel]["Wl"], lp[rel]["Wr"]], axis=0)
            return cat, w, lp[rel]["bl"]

        return ((cat_b, w_b, b_b),
                sage_dst(xg, "bus2gen", edge_index_bus_to_gen, ng, inv_b2g),
                sage_dst(xl, "bus2load", edge_index_bus_to_load, nl, inv_b2l),
                sage_dst(xs_, "bus2shunt", edge_index_bus_to_shunt, ns, inv_b2s))

    tb, tg, tl, ts = layer(x_bus, x_generator, x_load, x_shunt, params["l1"])
    h_b = _dense(*tb)
    h_g = _dense(*tg)
    h_l = _dense(*tl)
    h_s = _dense(*ts)

    tb, tg, tl, ts = layer(h_b, h_g, h_l, h_s, params["l2"])
    w2 = jnp.pad(params["lin"]["W"], ((0, 0), (0, 124)))
    b2 = jnp.pad(params["lin"]["b"], (0, 124))
    y_b = _dense(*tb, w2, b2)[:, :4]
    y_g = _dense(*tg, w2, b2)[:, :4]
    y_l = _dense(*tl, w2, b2)[:, :4]
    y_s = _dense(*ts, w2, b2)[:, :4]
    return (y_b, y_g, y_l, y_s)
